# baseline (speedup 1.0000x reference)
import functools

import jax
import jax.numpy as jnp
from jax.experimental import pallas as pl
from jax.experimental.pallas import tpu as pltpu

B, P, ZDIM, K = 16, 1024, 256, 16


def _silu(x):
    return x * jax.nn.sigmoid(x)


def _mlp_bn_silu(x, W, b):
    y = x @ W + b
    mu = jnp.mean(y, axis=0, keepdims=True)
    var = jnp.var(y, axis=0, keepdims=True)
    y = (y - mu) / jnp.sqrt(var + 1e-5)
    return jax.nn.silu(y)


def _knn_idx(pos, k):
    d2 = jnp.sum((pos[:, :, None, :] - pos[:, None, :, :]) ** 2, axis=-1)
    _, idx = jax.lax.top_k(-d2, k)
    return idx


def _pointnet_conv(x, pos, Wl, bl, Wg, bg, k):
    b, p, c = x.shape
    idx = _knn_idx(pos, k)
    bi = jnp.arange(b)[:, None, None]
    xj = x[bi, idx]
    pj = pos[bi, idx]
    rel = pj - pos[:, :, None, :]
    feat = jnp.concatenate([xj, rel], axis=-1).reshape(b * p * k, c + 3)
    msg = _mlp_bn_silu(feat, Wl, bl)
    gate = _mlp_bn_silu(msg, Wg, bg)
    oc = Wl.shape[1]
    msg = msg.reshape(b, p, k, oc)
    gate = gate.reshape(b, p, k, 1)
    alpha = jax.nn.softmax(gate, axis=2)
    return jnp.sum(alpha * msg, axis=2)


def _fps(pos, m):
    b, p, _ = pos.shape
    sel = jnp.zeros((b, m), dtype=jnp.int32)
    dist = jnp.sum((pos - pos[:, 0:1, :]) ** 2, axis=-1)

    def body(i, carry):
        sel, dist = carry
        nxt = jnp.argmax(dist, axis=1).astype(jnp.int32)
        sel = sel.at[:, i].set(nxt)
        npt = pos[jnp.arange(b), nxt]
        d = jnp.sum((pos - npt[:, None, :]) ** 2, axis=-1)
        return sel, jnp.minimum(dist, d)

    sel, _ = jax.lax.fori_loop(1, m, body, (sel, dist))
    return sel


def _final_pool_kernel(h_ref, wf_ref, bf_ref, out_ref):
    n = h_ref.shape[1]
    h = h_ref[...].reshape(B * n, ZDIM)
    y = jnp.dot(h, wf_ref[...], preferred_element_type=jnp.float32) + bf_ref[...]
    mu = jnp.mean(y, axis=0, keepdims=True)
    var = jnp.mean(y * y, axis=0, keepdims=True) - mu * mu
    g = _silu((y - mu) * jax.lax.rsqrt(var + 1e-5))
    g = g.reshape(B, n, ZDIM)
    g = g - jnp.max(g, axis=1, keepdims=True)
    e = jnp.exp(g)
    alpha = e / jnp.sum(e, axis=1, keepdims=True)
    out_ref[...] = jnp.sum(alpha * h.reshape(B, n, ZDIM), axis=1)


def _final_pool(h, Wf, bf):
    return pl.pallas_call(
        _final_pool_kernel,
        out_shape=jax.ShapeDtypeStruct((B, ZDIM), jnp.float32),
    )(h, Wf, bf)


def kernel(pos, batch, W1, b1, Wg1, bg1, W2, b2, Wg2, bg2, W3, b3, Wg3, bg3,
           Wf, bf):
    p0 = pos.reshape(B, P, 3)
    h = _pointnet_conv(p0, p0, W1, b1, Wg1, bg1, K)
    bi = jnp.arange(B)[:, None]
    sel = _fps(p0, P // 2)
    h, p1 = h[bi, sel], p0[bi, sel]
    h = _pointnet_conv(h, p1, W2, b2, Wg2, bg2, K)
    sel = _fps(p1, P // 4)
    h, p2 = h[bi, sel], p1[bi, sel]
    h = _pointnet_conv(h, p2, W3, b3, Wg3, bg3, K)
    return _final_pool(h, Wf, bf)


# speedup vs baseline: 7.6409x; 7.6409x over previous
import jax
import jax.numpy as jnp
from jax.experimental import pallas as pl
from jax.experimental.pallas import tpu as pltpu

B, P, ZDIM, K = 16, 1024, 256, 16
EPS = 1e-5
BIG = 3.0e38
CP = pltpu.CompilerParams(vmem_limit_bytes=110 * 1024 * 1024)


def _silu(x):
    return x * jax.nn.sigmoid(x)


def _bf16_split3(mat):
    hi = mat.astype(jnp.bfloat16).astype(jnp.float32)
    r = mat - hi
    mid = r.astype(jnp.bfloat16).astype(jnp.float32)
    lo = r - mid
    return hi, mid, lo


def _exact_gather(oh, parts):
    hi, mid, lo = parts
    return (jnp.dot(oh, hi, preferred_element_type=jnp.float32)
            + jnp.dot(oh, mid, preferred_element_type=jnp.float32)
            + jnp.dot(oh, lo, preferred_element_type=jnp.float32))


def _convA_kernel(x_ref, pos_ref, w_ref, b_ref, y_ref, s1_ref, s2_ref):
    g = pl.program_id(0)
    x = x_ref[0]
    pos = pos_ref[0]
    p = x.shape[0]
    cin = x.shape[1]
    oc = w_ref.shape[1]
    w = w_ref[...]
    bb = b_ref[...]
    cat = jnp.concatenate([x, pos], axis=1)
    catp = _bf16_split3(cat)
    z = jnp.concatenate([jnp.zeros_like(x), pos], axis=1)
    pt = pos.T
    d2 = jnp.zeros((p, p), jnp.float32)
    for c in range(3):
        diff = pos[:, c:c + 1] - pt[c:c + 1, :]
        d2 = d2 + diff * diff
    lane = jax.lax.broadcasted_iota(jnp.int32, (p, p), 1)
    s1 = jnp.zeros((1, oc), jnp.float32)
    s2 = jnp.zeros((1, oc), jnp.float32)
    for t in range(K):
        mn = jnp.min(d2, axis=1, keepdims=True)
        eq = d2 <= mn
        idx = jnp.min(jnp.where(eq, lane, p), axis=1, keepdims=True)
        oh = (lane == idx)
        catj = _exact_gather(oh.astype(jnp.float32), catp)
        y_t = jnp.dot(catj - z, w,
                      preferred_element_type=jnp.float32) + bb
        y_ref[0, :, t, :] = y_t
        s1 = s1 + jnp.sum(y_t, axis=0, keepdims=True)
        s2 = s2 + jnp.sum(y_t * y_t, axis=0, keepdims=True)
        d2 = jnp.where(oh, BIG, d2)

    @pl.when(g == 0)
    def _():
        s1_ref[...] = jnp.zeros_like(s1_ref)
        s2_ref[...] = jnp.zeros_like(s2_ref)

    s1_ref[...] += s1
    s2_ref[...] += s2


def _convA(x, pos, w, b):
    bn, p, cin = x.shape
    oc = w.shape[1]
    y, s1, s2 = pl.pallas_call(
        _convA_kernel,
        grid=(bn,),
        in_specs=[
            pl.BlockSpec((1, p, cin), lambda i: (i, 0, 0)),
            pl.BlockSpec((1, p, 3), lambda i: (i, 0, 0)),
            pl.BlockSpec((cin + 3, oc), lambda i: (0, 0)),
            pl.BlockSpec((1, oc), lambda i: (0, 0)),
        ],
        out_specs=[
            pl.BlockSpec((1, p, K, oc), lambda i: (i, 0, 0, 0)),
            pl.BlockSpec((1, oc), lambda i: (0, 0)),
            pl.BlockSpec((1, oc), lambda i: (0, 0)),
        ],
        out_shape=[
            jax.ShapeDtypeStruct((bn, p, K, oc), jnp.float32),
            jax.ShapeDtypeStruct((1, oc), jnp.float32),
            jax.ShapeDtypeStruct((1, oc), jnp.float32),
        ],
        compiler_params=CP,
    )(x, pos, w, b.reshape(1, oc))
    n = bn * p * K
    mu = s1 / n
    var = s2 / n - mu * mu
    isig = jax.lax.rsqrt(var + EPS)
    return y, mu, isig


def _convB_kernel(y_ref, mu_ref, isig_ref, wg_ref, sg_ref, sg2_ref):
    g = pl.program_id(0)
    y = y_ref[0]
    msg = _silu((y - mu_ref[...][None]) * isig_ref[...][None])
    msgb = msg.astype(jnp.bfloat16).astype(jnp.float32)
    wgb = wg_ref[...][None].astype(jnp.bfloat16).astype(jnp.float32)
    mw = jnp.sum(msgb * wgb, axis=2)
    s = jnp.sum(mw)
    s2 = jnp.sum(mw * mw)

    @pl.when(g == 0)
    def _():
        sg_ref[...] = jnp.zeros_like(sg_ref)
        sg2_ref[...] = jnp.zeros_like(sg2_ref)

    sg_ref[...] += jnp.full(sg_ref.shape, s, jnp.float32)
    sg2_ref[...] += jnp.full(sg2_ref.shape, s2, jnp.float32)


def _convB(y, mu, isig, wg, bg):
    bn, p, k, oc = y.shape
    sg, sg2 = pl.pallas_call(
        _convB_kernel,
        grid=(bn,),
        in_specs=[
            pl.BlockSpec((1, p, k, oc), lambda i: (i, 0, 0, 0)),
            pl.BlockSpec((1, oc), lambda i: (0, 0)),
            pl.BlockSpec((1, oc), lambda i: (0, 0)),
            pl.BlockSpec((1, oc), lambda i: (0, 0)),
        ],
        out_specs=[
            pl.BlockSpec((1, 128), lambda i: (0, 0)),
            pl.BlockSpec((1, 128), lambda i: (0, 0)),
        ],
        out_shape=[
            jax.ShapeDtypeStruct((1, 128), jnp.float32),
            jax.ShapeDtypeStruct((1, 128), jnp.float32),
        ],
        compiler_params=CP,
    )(y, mu, isig, wg)
    n = bn * p * k
    smw = sg[0, 0]
    smw2 = sg2[0, 0]
    gmu = smw / n + bg[0]
    gvar = smw2 / n - (smw / n) ** 2
    gisig = jax.lax.rsqrt(gvar + EPS)
    ga = jnp.full((1, 128), gisig, jnp.float32)
    gc = jnp.full((1, 128), (gmu - bg[0]) * gisig, jnp.float32)
    return ga, gc


def _convC_kernel(y_ref, mu_ref, isig_ref, wg_ref, ga_ref, gc_ref, h_ref):
    y = y_ref[0]
    msg = _silu((y - mu_ref[...][None]) * isig_ref[...][None])
    msgb = msg.astype(jnp.bfloat16).astype(jnp.float32)
    wgb = wg_ref[...][None].astype(jnp.bfloat16).astype(jnp.float32)
    mw = jnp.sum(msgb * wgb, axis=2)
    ghat = _silu(mw * ga_ref[0:1, 0:1] - gc_ref[0:1, 0:1])
    ghat = ghat - jnp.max(ghat, axis=1, keepdims=True)
    e = jnp.exp(ghat)
    alpha = e / jnp.sum(e, axis=1, keepdims=True)
    h_ref[0] = jnp.sum(alpha[:, :, None] * msg, axis=1)


def _convC(y, mu, isig, wg, ga, gc):
    bn, p, k, oc = y.shape
    return pl.pallas_call(
        _convC_kernel,
        grid=(bn,),
        in_specs=[
            pl.BlockSpec((1, p, k, oc), lambda i: (i, 0, 0, 0)),
            pl.BlockSpec((1, oc), lambda i: (0, 0)),
            pl.BlockSpec((1, oc), lambda i: (0, 0)),
            pl.BlockSpec((1, oc), lambda i: (0, 0)),
            pl.BlockSpec((1, 128), lambda i: (0, 0)),
            pl.BlockSpec((1, 128), lambda i: (0, 0)),
        ],
        out_specs=pl.BlockSpec((1, p, oc), lambda i: (i, 0, 0)),
        out_shape=jax.ShapeDtypeStruct((bn, p, oc), jnp.float32),
        compiler_params=CP,
    )(y, mu, isig, wg, ga, gc)


def _conv_layer(x, pos, w, b, wg, bg):
    y, mu, isig = _convA(x, pos, w, b)
    wgr = wg.reshape(1, -1)
    ga, gc = _convB(y, mu, isig, wgr, bg)
    return _convC(y, mu, isig, wgr, ga, gc)


def _fps_kernel(post_ref, pos_ref, h_ref, psub_ref, hsub_ref, oh_ref, m):
    bn = post_ref.shape[0]
    p = post_ref.shape[2]
    oc = h_ref.shape[2]
    lane = jax.lax.broadcasted_iota(jnp.int32, (bn, p), 1)
    oh_ref[:, 0:1, :] = (lane == 0).astype(jnp.float32)[:, None, :]
    d0 = jnp.zeros((bn, p), jnp.float32)
    for c in range(3):
        pc = post_ref[:, c, :]
        d0 = d0 + (pc - pc[:, 0:1]) ** 2

    def body(i, dist):
        mx = jnp.max(dist, axis=1, keepdims=True)
        eq = dist >= mx
        idx = jnp.min(jnp.where(eq, lane, p), axis=1, keepdims=True)
        oh = (lane == idx).astype(jnp.float32)
        oh_ref[:, pl.ds(i, 1), :] = oh[:, None, :]
        d = jnp.zeros((bn, p), jnp.float32)
        for c in range(3):
            pc = post_ref[:, c, :]
            nc = jnp.sum(oh * pc, axis=1, keepdims=True)
            d = d + (pc - nc) ** 2
        return jnp.minimum(dist, d)

    jax.lax.fori_loop(1, m, body, d0)
    for bb in range(B):
        ohb = oh_ref[bb]
        psub_ref[bb] = _exact_gather(ohb, _bf16_split3(pos_ref[bb]))
        hsub_ref[bb] = _exact_gather(ohb, _bf16_split3(h_ref[bb]))


def _fps_subsample(pos, h, m):
    bn, p, oc = h.shape
    post = pos.transpose(0, 2, 1)
    import functools
    psub, hsub = pl.pallas_call(
        functools.partial(_fps_kernel, m=m),
        in_specs=[
            pl.BlockSpec((bn, 3, p), lambda: (0, 0, 0)),
            pl.BlockSpec((bn, p, 3), lambda: (0, 0, 0)),
            pl.BlockSpec((bn, p, oc), lambda: (0, 0, 0)),
        ],
        out_specs=[
            pl.BlockSpec((bn, m, 3), lambda: (0, 0, 0)),
            pl.BlockSpec((bn, m, oc), lambda: (0, 0, 0)),
        ],
        out_shape=[
            jax.ShapeDtypeStruct((bn, m, 3), jnp.float32),
            jax.ShapeDtypeStruct((bn, m, oc), jnp.float32),
        ],
        scratch_shapes=[pltpu.VMEM((bn, m, p), jnp.float32)],
        compiler_params=CP,
    )(post, pos, h)
    return psub, hsub


def _final_pool_kernel(h_ref, wf_ref, bf_ref, out_ref):
    n = h_ref.shape[1]
    h = h_ref[...].reshape(B * n, ZDIM)
    y = jnp.dot(h, wf_ref[...], preferred_element_type=jnp.float32) + bf_ref[...]
    mu = jnp.mean(y, axis=0, keepdims=True)
    var = jnp.mean(y * y, axis=0, keepdims=True) - mu * mu
    g = _silu((y - mu) * jax.lax.rsqrt(var + EPS))
    g = g.reshape(B, n, ZDIM)
    g = g - jnp.max(g, axis=1, keepdims=True)
    e = jnp.exp(g)
    alpha = e / jnp.sum(e, axis=1, keepdims=True)
    out_ref[...] = jnp.sum(alpha * h.reshape(B, n, ZDIM), axis=1)


def _final_pool(h, wf, bf):
    return pl.pallas_call(
        _final_pool_kernel,
        out_shape=jax.ShapeDtypeStruct((B, ZDIM), jnp.float32),
        compiler_params=CP,
    )(h, wf, bf.reshape(1, ZDIM))


def kernel(pos, batch, W1, b1, Wg1, bg1, W2, b2, Wg2, bg2, W3, b3, Wg3, bg3,
           Wf, bf):
    p0 = pos.reshape(B, P, 3)
    h = _conv_layer(p0, p0, W1, b1, Wg1, bg1)
    p1, h = _fps_subsample(p0, h, P // 2)
    h = _conv_layer(h, p1, W2, b2, Wg2, bg2)
    p2, h = _fps_subsample(p1, h, P // 4)
    h = _conv_layer(h, p2, W3, b3, Wg3, bg3)
    return _final_pool(h, Wf, bf)


# speedup vs baseline: 8.4705x; 1.1086x over previous
import jax
import jax.numpy as jnp
from jax.experimental import pallas as pl
from jax.experimental.pallas import tpu as pltpu

B, P, ZDIM, K = 16, 1024, 256, 16
EPS = 1e-5
BIG = 3.0e38
CP = pltpu.CompilerParams(vmem_limit_bytes=110 * 1024 * 1024)


def _silu(x):
    return x * jax.nn.sigmoid(x)


def _bf16_split3(mat):
    hi = mat.astype(jnp.bfloat16).astype(jnp.float32)
    r = mat - hi
    mid = r.astype(jnp.bfloat16).astype(jnp.float32)
    lo = r - mid
    return hi, mid, lo


def _exact_gather(oh, parts):
    hi, mid, lo = parts
    return (jnp.dot(oh, hi, preferred_element_type=jnp.float32)
            + jnp.dot(oh, mid, preferred_element_type=jnp.float32)
            + jnp.dot(oh, lo, preferred_element_type=jnp.float32))


def _convA_kernel(x_ref, pos_ref, w_ref, b_ref, y_ref, s1_ref, s2_ref):
    g = pl.program_id(0)
    x = x_ref[0]
    pos = pos_ref[0]
    p = x.shape[0]
    cin = x.shape[1]
    oc = w_ref.shape[1]
    w = w_ref[...]
    bb = b_ref[...]
    cat = jnp.concatenate([x, pos], axis=1)
    catp = jnp.concatenate(_bf16_split3(cat), axis=1)
    z = jnp.concatenate([jnp.zeros_like(x), pos], axis=1)
    pt = pos.T
    d2 = jnp.zeros((p, p), jnp.float32)
    for c in range(3):
        diff = pos[:, c:c + 1] - pt[c:c + 1, :]
        d2 = d2 + diff * diff
    lane = jax.lax.broadcasted_iota(jnp.int32, (p, p), 1)
    s1 = jnp.zeros((1, oc), jnp.float32)
    s2 = jnp.zeros((1, oc), jnp.float32)
    for t in range(K):
        mn = jnp.min(d2, axis=1, keepdims=True)
        eq = d2 <= mn
        idx = jnp.min(jnp.where(eq, lane, p), axis=1, keepdims=True)
        oh = (lane == idx)
        cj3 = jnp.dot(oh.astype(jnp.float32), catp,
                      preferred_element_type=jnp.float32)
        cw = cat.shape[1]
        catj = cj3[:, :cw] + cj3[:, cw:2 * cw] + cj3[:, 2 * cw:]
        y_t = jnp.dot(catj - z, w,
                      preferred_element_type=jnp.float32) + bb
        y_ref[0, :, t, :] = y_t
        s1 = s1 + jnp.sum(y_t, axis=0, keepdims=True)
        s2 = s2 + jnp.sum(y_t * y_t, axis=0, keepdims=True)
        d2 = jnp.where(oh, BIG, d2)

    @pl.when(g == 0)
    def _():
        s1_ref[...] = jnp.zeros_like(s1_ref)
        s2_ref[...] = jnp.zeros_like(s2_ref)

    s1_ref[...] += s1
    s2_ref[...] += s2


def _convA(x, pos, w, b):
    bn, p, cin = x.shape
    oc = w.shape[1]
    y, s1, s2 = pl.pallas_call(
        _convA_kernel,
        grid=(bn,),
        in_specs=[
            pl.BlockSpec((1, p, cin), lambda i: (i, 0, 0)),
            pl.BlockSpec((1, p, 3), lambda i: (i, 0, 0)),
            pl.BlockSpec((cin + 3, oc), lambda i: (0, 0)),
            pl.BlockSpec((1, oc), lambda i: (0, 0)),
        ],
        out_specs=[
            pl.BlockSpec((1, p, K, oc), lambda i: (i, 0, 0, 0)),
            pl.BlockSpec((1, oc), lambda i: (0, 0)),
            pl.BlockSpec((1, oc), lambda i: (0, 0)),
        ],
        out_shape=[
            jax.ShapeDtypeStruct((bn, p, K, oc), jnp.float32),
            jax.ShapeDtypeStruct((1, oc), jnp.float32),
            jax.ShapeDtypeStruct((1, oc), jnp.float32),
        ],
        compiler_params=CP,
    )(x, pos, w, b.reshape(1, oc))
    n = bn * p * K
    mu = s1 / n
    var = s2 / n - mu * mu
    isig = jax.lax.rsqrt(var + EPS)
    return y, mu, isig


def _convB_kernel(y_ref, mu_ref, isig_ref, wg_ref, sg_ref, sg2_ref):
    g = pl.program_id(0)
    y = y_ref[0]
    msg = _silu((y - mu_ref[...][None]) * isig_ref[...][None])
    msgb = msg.astype(jnp.bfloat16).astype(jnp.float32)
    wgb = wg_ref[...][None].astype(jnp.bfloat16).astype(jnp.float32)
    mw = jnp.sum(msgb * wgb, axis=2)
    s = jnp.sum(mw)
    s2 = jnp.sum(mw * mw)

    @pl.when(g == 0)
    def _():
        sg_ref[...] = jnp.zeros_like(sg_ref)
        sg2_ref[...] = jnp.zeros_like(sg2_ref)

    sg_ref[...] += jnp.full(sg_ref.shape, s, jnp.float32)
    sg2_ref[...] += jnp.full(sg2_ref.shape, s2, jnp.float32)


def _convB(y, mu, isig, wg, bg):
    bn, p, k, oc = y.shape
    sg, sg2 = pl.pallas_call(
        _convB_kernel,
        grid=(bn,),
        in_specs=[
            pl.BlockSpec((1, p, k, oc), lambda i: (i, 0, 0, 0)),
            pl.BlockSpec((1, oc), lambda i: (0, 0)),
            pl.BlockSpec((1, oc), lambda i: (0, 0)),
            pl.BlockSpec((1, oc), lambda i: (0, 0)),
        ],
        out_specs=[
            pl.BlockSpec((1, 128), lambda i: (0, 0)),
            pl.BlockSpec((1, 128), lambda i: (0, 0)),
        ],
        out_shape=[
            jax.ShapeDtypeStruct((1, 128), jnp.float32),
            jax.ShapeDtypeStruct((1, 128), jnp.float32),
        ],
        compiler_params=CP,
    )(y, mu, isig, wg)
    n = bn * p * k
    smw = sg[0, 0]
    smw2 = sg2[0, 0]
    gmu = smw / n + bg[0]
    gvar = smw2 / n - (smw / n) ** 2
    gisig = jax.lax.rsqrt(gvar + EPS)
    ga = jnp.full((1, 128), gisig, jnp.float32)
    gc = jnp.full((1, 128), (gmu - bg[0]) * gisig, jnp.float32)
    return ga, gc


def _convC_kernel(y_ref, mu_ref, isig_ref, wg_ref, ga_ref, gc_ref, h_ref):
    y = y_ref[0]
    msg = _silu((y - mu_ref[...][None]) * isig_ref[...][None])
    msgb = msg.astype(jnp.bfloat16).astype(jnp.float32)
    wgb = wg_ref[...][None].astype(jnp.bfloat16).astype(jnp.float32)
    mw = jnp.sum(msgb * wgb, axis=2)
    ghat = _silu(mw * ga_ref[0:1, 0:1] - gc_ref[0:1, 0:1])
    ghat = ghat - jnp.max(ghat, axis=1, keepdims=True)
    e = jnp.exp(ghat)
    alpha = e / jnp.sum(e, axis=1, keepdims=True)
    h_ref[0] = jnp.sum(alpha[:, :, None] * msg, axis=1)


def _convC(y, mu, isig, wg, ga, gc):
    bn, p, k, oc = y.shape
    return pl.pallas_call(
        _convC_kernel,
        grid=(bn,),
        in_specs=[
            pl.BlockSpec((1, p, k, oc), lambda i: (i, 0, 0, 0)),
            pl.BlockSpec((1, oc), lambda i: (0, 0)),
            pl.BlockSpec((1, oc), lambda i: (0, 0)),
            pl.BlockSpec((1, oc), lambda i: (0, 0)),
            pl.BlockSpec((1, 128), lambda i: (0, 0)),
            pl.BlockSpec((1, 128), lambda i: (0, 0)),
        ],
        out_specs=pl.BlockSpec((1, p, oc), lambda i: (i, 0, 0)),
        out_shape=jax.ShapeDtypeStruct((bn, p, oc), jnp.float32),
        compiler_params=CP,
    )(y, mu, isig, wg, ga, gc)


def _conv_layer(x, pos, w, b, wg, bg):
    y, mu, isig = _convA(x, pos, w, b)
    wgr = wg.reshape(1, -1)
    ga, gc = _convB(y, mu, isig, wgr, bg)
    return _convC(y, mu, isig, wgr, ga, gc)


def _fps_kernel(post_ref, pos_ref, h_ref, psub_ref, hsub_ref, oh_ref, m):
    bn = post_ref.shape[0]
    p = post_ref.shape[2]
    oc = h_ref.shape[2]
    lane = jax.lax.broadcasted_iota(jnp.int32, (bn, p), 1)
    oh_ref[:, 0:1, :] = (lane == 0).astype(jnp.float32)[:, None, :]
    d0 = jnp.zeros((bn, p), jnp.float32)
    for c in range(3):
        pc = post_ref[:, c, :]
        d0 = d0 + (pc - pc[:, 0:1]) ** 2

    def body(i, dist):
        mx = jnp.max(dist, axis=1, keepdims=True)
        eq = dist >= mx
        idx = jnp.min(jnp.where(eq, lane, p), axis=1, keepdims=True)
        oh = (lane == idx).astype(jnp.float32)
        oh_ref[:, pl.ds(i, 1), :] = oh[:, None, :]
        d = jnp.zeros((bn, p), jnp.float32)
        for c in range(3):
            pc = post_ref[:, c, :]
            nc = jnp.sum(oh * pc, axis=1, keepdims=True)
            d = d + (pc - nc) ** 2
        return jnp.minimum(dist, d)

    jax.lax.fori_loop(1, m, body, d0)
    for bb in range(B):
        ohb = oh_ref[bb]
        psub_ref[bb] = _exact_gather(ohb, _bf16_split3(pos_ref[bb]))
        hsub_ref[bb] = _exact_gather(ohb, _bf16_split3(h_ref[bb]))


def _fps_subsample(pos, h, m):
    bn, p, oc = h.shape
    post = pos.transpose(0, 2, 1)
    import functools
    psub, hsub = pl.pallas_call(
        functools.partial(_fps_kernel, m=m),
        in_specs=[
            pl.BlockSpec((bn, 3, p), lambda: (0, 0, 0)),
            pl.BlockSpec((bn, p, 3), lambda: (0, 0, 0)),
            pl.BlockSpec((bn, p, oc), lambda: (0, 0, 0)),
        ],
        out_specs=[
            pl.BlockSpec((bn, m, 3), lambda: (0, 0, 0)),
            pl.BlockSpec((bn, m, oc), lambda: (0, 0, 0)),
        ],
        out_shape=[
            jax.ShapeDtypeStruct((bn, m, 3), jnp.float32),
            jax.ShapeDtypeStruct((bn, m, oc), jnp.float32),
        ],
        scratch_shapes=[pltpu.VMEM((bn, m, p), jnp.float32)],
        compiler_params=CP,
    )(post, pos, h)
    return psub, hsub


def _final_pool_kernel(h_ref, wf_ref, bf_ref, out_ref):
    n = h_ref.shape[1]
    h = h_ref[...].reshape(B * n, ZDIM)
    y = jnp.dot(h, wf_ref[...], preferred_element_type=jnp.float32) + bf_ref[...]
    mu = jnp.mean(y, axis=0, keepdims=True)
    var = jnp.mean(y * y, axis=0, keepdims=True) - mu * mu
    g = _silu((y - mu) * jax.lax.rsqrt(var + EPS))
    g = g.reshape(B, n, ZDIM)
    g = g - jnp.max(g, axis=1, keepdims=True)
    e = jnp.exp(g)
    alpha = e / jnp.sum(e, axis=1, keepdims=True)
    out_ref[...] = jnp.sum(alpha * h.reshape(B, n, ZDIM), axis=1)


def _final_pool(h, wf, bf):
    return pl.pallas_call(
        _final_pool_kernel,
        out_shape=jax.ShapeDtypeStruct((B, ZDIM), jnp.float32),
        compiler_params=CP,
    )(h, wf, bf.reshape(1, ZDIM))


def kernel(pos, batch, W1, b1, Wg1, bg1, W2, b2, Wg2, bg2, W3, b3, Wg3, bg3,
           Wf, bf):
    p0 = pos.reshape(B, P, 3)
    h = _conv_layer(p0, p0, W1, b1, Wg1, bg1)
    p1, h = _fps_subsample(p0, h, P // 2)
    h = _conv_layer(h, p1, W2, b2, Wg2, bg2)
    p2, h = _fps_subsample(p1, h, P // 4)
    h = _conv_layer(h, p2, W3, b3, Wg3, bg3)
    return _final_pool(h, Wf, bf)
